# baseline (speedup 1.0000x reference)
import jax
import jax.numpy as jnp
import numpy as np
from jax import lax
from jax.experimental import pallas as pl
from jax.experimental.pallas import tpu as pltpu
from jax.experimental.pallas import tpu_sc as plsc

E = 8
K = 4
D_ENC = 256
D_LAT = 256
HID = 256
OUT = 4
DEPTH = 7
SKIP = 5
CH = D_LAT // E
EIN = D_ENC + CH
XL = D_ENC + D_LAT

T = 4096
TB = 512
B = 256
LOGB = 8
NB = T // B + E
NBP = 32
P = NB * B
FW = 128

NS = 16
TPW = T // NS
SPW = P // NS
QR = SPW // 4
L = 16

_EPS = np.float32(np.finfo(np.float32).eps)


def _dot(a, b):
    return jnp.dot(a, b, preferred_element_type=jnp.float32)


def _gate_body(x_ref, lat_ref, gW1, gb1, gW2, gb2, gln_g, gln_b, gW3, gb3,
               eidx_ref, gval_ref):
    xb = x_ref[...]
    lb = lat_ref[...]
    g = jax.nn.relu(_dot(xb, gW1[:D_ENC]) + _dot(lb, gW1[D_ENC:]) + gb1[...])
    g = _dot(g, gW2[...]) + gb2[...]
    m = g.mean(-1, keepdims=True)
    v = ((g - m) ** 2).mean(-1, keepdims=True)
    g = (g - m) / jnp.sqrt(v + 1e-5) * gln_g[...] + gln_b[...]
    logits = _dot(g, gW3[...]) + gb3[...]
    mx = logits.max(-1, keepdims=True)
    s = jnp.exp(logits - mx)
    s = s / s.sum(-1, keepdims=True)
    lanes = jax.lax.broadcasted_iota(jnp.int32, s.shape, 1)
    cur = s
    sum4 = jnp.zeros((s.shape[0], 1), jnp.float32)
    eidx = gmax = None
    for r in range(K):
        m_r = cur.max(-1, keepdims=True)
        i_r = jnp.where(cur == m_r, lanes, E).min(-1, keepdims=True)
        sum4 = sum4 + m_r
        if r == 0:
            eidx, gmax = i_r, m_r
        cur = jnp.where(lanes == i_r, -jnp.inf, cur)
    gval = gmax / (sum4 + 1e-9)
    eidx_ref[...] = eidx.reshape(1, TB, 1)
    gval_ref[...] = gval.reshape(1, TB, 1)


def _gate(x, latent, gW1, gb1, gW2, gb2, gln_g, gln_b, gW3, gb3):
    tok = lambda i: (i, 0)
    out3 = lambda i: (i, 0, 0)
    def wspec(a):
        return pl.BlockSpec(a.shape, lambda i, _a=a: tuple([0] * _a.ndim))
    eidx, gval = pl.pallas_call(
        _gate_body,
        grid=(T // TB,),
        in_specs=[pl.BlockSpec((TB, D_ENC), tok),
                  pl.BlockSpec((TB, D_LAT), tok)]
                 + [wspec(a) for a in (gW1, gb1, gW2, gb2, gln_g, gln_b,
                                       gW3, gb3)],
        out_specs=[pl.BlockSpec((1, TB, 1), out3),
                   pl.BlockSpec((1, TB, 1), out3)],
        out_shape=[jax.ShapeDtypeStruct((T // TB, TB, 1), jnp.int32),
                   jax.ShapeDtypeStruct((T // TB, TB, 1), jnp.float32)],
    )(x, latent, gW1, gb1, gW2, gb2, gln_g, gln_b, gW3, gb3)
    return eidx.reshape(T), gval.reshape(T)


def _route_body(eidx_h, gval_h, xl_h,
                xls_h, gs_h, slots_h, bexp_h,
                ids_v, slots_v, cnt_v, cnt_all_v, off_s, allslots_v,
                perm_v, bexp_v, gval_v, gs_v, rows_v,
                cnt_sh, slots_sh, sem):
    wid = lax.axis_index("s")
    tbase = wid * TPW
    sbase = wid * SPW

    pltpu.sync_copy(eidx_h.at[pl.ds(tbase, TPW)], ids_v)
    lane = lax.iota(jnp.int32, L)
    cnt = jnp.zeros((L,), jnp.int32)
    for k in range(TPW // L):
        v = ids_v[pl.ds(k * L, L)]
        for e in range(E):
            pc = plsc.all_reduce_population_count(v == e)
            cnt = cnt + jnp.where(lane == e, pc, 0)
    cnt_v[...] = cnt
    pltpu.sync_copy(cnt_v, cnt_sh.at[wid])
    plsc.subcore_barrier()

    pltpu.sync_copy(cnt_sh, cnt_all_v)
    total = jnp.zeros((L,), jnp.int32)
    mybase = jnp.zeros((L,), jnp.int32)
    for w in range(NS):
        row = cnt_all_v[w]
        total = total + row
        mybase = mybase + jnp.where(wid > w, row, 0)
    padded = ((total + (B - 1)) >> LOGB) << LOGB
    cum = plsc.cumsum(padded)
    astart = cum - padded
    off = astart + mybase
    for e in range(E):
        off_s[e] = off[e]

    for k in range(TPW // L):
        v = ids_v[pl.ds(k * L, L)]
        sv = jnp.zeros((L,), jnp.int32)
        for j in range(L):
            e = v[j]
            o = off_s[e]
            off_s[e] = o + 1
            sv = jnp.where(lane == j, o, sv)
        slots_v[pl.ds(k * L, L)] = sv
    pltpu.sync_copy(slots_v, slots_h.at[pl.ds(tbase, TPW)])
    pltpu.sync_copy(slots_v, slots_sh.at[pl.ds(tbase, TPW)])
    plsc.subcore_barrier()

    for k in range(NBP // L):
        bvec = (lane + k * L) << LOGB
        acc = jnp.zeros((L,), jnp.int32)
        for e in range(E):
            acc = acc + jnp.where(bvec >= cum[e], 1, 0)
        bexp_v[pl.ds(k * L, L)] = jnp.minimum(acc, E - 1)
    @pl.when(wid == 0)
    def _():
        pltpu.sync_copy(bexp_v, bexp_h)

    pltpu.sync_copy(slots_sh, allslots_v)
    for k in range(SPW // L):
        perm_v[pl.ds(k * L, L)] = jnp.zeros((L,), jnp.int32)
    def scat(k, _):
        s = allslots_v[pl.ds(k * L, L)]
        toks = k * L + lane
        mask = (s >= sbase) & (s < sbase + SPW)
        plsc.store_scatter(perm_v, [s - sbase], toks, mask=mask)
        return 0
    lax.fori_loop(0, T // L, scat, 0)

    pltpu.sync_copy(gval_h, gval_v)
    for k in range(SPW // L):
        pv = perm_v[pl.ds(k * L, L)]
        gs_v[pl.ds(k * L, L)] = plsc.load_gather(gval_v, [pv])
    pltpu.sync_copy(gs_v, gs_h.at[pl.ds(sbase, SPW)])

    for q in range(SPW // QR):
        pltpu.async_copy(xl_h.at[perm_v.at[pl.ds(q * QR, QR)]],
                         rows_v, sem).wait()
        pltpu.sync_copy(rows_v, xls_h.at[pl.ds(sbase + q * QR, QR), :])


def _route(eidx, gval, xl):
    mesh = plsc.VectorSubcoreMesh(core_axis_name="c", subcore_axis_name="s",
                                  num_cores=1)
    f = pl.kernel(
        _route_body,
        compiler_params=pltpu.CompilerParams(needs_layout_passes=False),
        out_type=[jax.ShapeDtypeStruct((P, XL), jnp.float32),
                  jax.ShapeDtypeStruct((P,), jnp.float32),
                  jax.ShapeDtypeStruct((T,), jnp.int32),
                  jax.ShapeDtypeStruct((NBP,), jnp.int32)],
        mesh=mesh,
        scratch_types=[
            pltpu.VMEM((TPW,), jnp.int32),
            pltpu.VMEM((TPW,), jnp.int32),
            pltpu.VMEM((L,), jnp.int32),
            pltpu.VMEM((NS, L), jnp.int32),
            pltpu.SMEM((E,), jnp.int32),
            pltpu.VMEM((T,), jnp.int32),
            pltpu.VMEM((SPW,), jnp.int32),
            pltpu.VMEM((NBP,), jnp.int32),
            pltpu.VMEM((T,), jnp.float32),
            pltpu.VMEM((SPW,), jnp.float32),
            pltpu.VMEM((QR, XL), jnp.float32),
            pltpu.VMEM_SHARED((NS, L), jnp.int32),
            pltpu.VMEM_SHARED((T,), jnp.int32),
            pltpu.SemaphoreType.DMA,
        ],
    )
    return f(eidx, gval, xl)


def _mlp_body(bexp_sm, xls_ref, gs_ref, eW0, eb0, eWh, ebh, eWs, ebs,
              eWo, ebo, out_ref):
    e = bexp_sm[pl.program_id(0)]
    xb = xls_ref[:, :D_ENC]
    latb = xls_ref[:, D_ENC:]
    sel = (jax.lax.broadcasted_iota(jnp.int32, (D_LAT, CH), 0)
           == e * CH + jax.lax.broadcasted_iota(jnp.int32, (D_LAT, CH), 1)
           ).astype(jnp.float32)
    chunk = _dot(latb, sel)
    h0 = jnp.concatenate([xb, chunk], axis=-1)
    h = jax.nn.relu(_dot(h0, eW0[0]) + eb0[0])
    hidx = 0
    for i in range(1, DEPTH):
        if i == SKIP:
            h = jnp.concatenate([h, h0], axis=-1)
            h = jax.nn.relu(_dot(h, eWs[0]) + ebs[0])
        else:
            h = jax.nn.relu(_dot(h, eWh[0, hidx]) + ebh[0, hidx])
            hidx += 1
    o = _dot(h, eWo[0]) + ebo[0]
    c = jnp.exp(o) * gs_ref[...]
    c = jnp.where(c == 0, _EPS, c)
    res = jnp.log(c)
    out_ref[...] = jnp.concatenate(
        [res, jnp.zeros((B, FW - OUT), jnp.float32)], axis=-1)


def _grouped_mlp(bexp, xls, gs, eW0, eb0, eWh, ebh, eWs, ebs, eWo, ebo):
    def ws(a):
        nd = a.ndim - 1
        return pl.BlockSpec((1,) + a.shape[1:],
                            lambda i, be, _n=nd: (be[i],) + (0,) * _n)
    grid_spec = pltpu.PrefetchScalarGridSpec(
        num_scalar_prefetch=1,
        grid=(NB,),
        in_specs=[
            pl.BlockSpec((B, XL), lambda i, be: (i, 0)),
            pl.BlockSpec((B, 1), lambda i, be: (i, 0)),
            ws(eW0), ws(eb0), ws(eWh), ws(ebh),
            ws(eWs), ws(ebs), ws(eWo), ws(ebo),
        ],
        out_specs=pl.BlockSpec((B, FW), lambda i, be: (i, 0)),
    )
    return pl.pallas_call(
        _mlp_body,
        grid_spec=grid_spec,
        out_shape=jax.ShapeDtypeStruct((P, FW), jnp.float32),
    )(bexp, xls, gs, eW0, eb0, eWh, ebh, eWs, ebs, eWo, ebo)


def _combine_body(fs_h, slots_h, out_h, myslots_v, rows_v, sem):
    wid = lax.axis_index("s")
    tbase = wid * TPW
    pltpu.sync_copy(slots_h.at[pl.ds(tbase, TPW)], myslots_v)
    pltpu.async_copy(fs_h.at[myslots_v], rows_v, sem).wait()
    pltpu.sync_copy(rows_v, out_h.at[pl.ds(tbase, TPW), :])


def _combine(fs, slots):
    mesh = plsc.VectorSubcoreMesh(core_axis_name="c", subcore_axis_name="s",
                                  num_cores=1)
    f = pl.kernel(
        _combine_body,
        compiler_params=pltpu.CompilerParams(needs_layout_passes=False),
        out_type=jax.ShapeDtypeStruct((T, FW), jnp.float32),
        mesh=mesh,
        scratch_types=[
            pltpu.VMEM((TPW,), jnp.int32),
            pltpu.VMEM((TPW, FW), jnp.float32),
            pltpu.SemaphoreType.DMA,
        ],
    )
    return f(fs, slots)


def kernel(x, latent, gW1, gb1, gW2, gb2, gln_g, gln_b, gW3, gb3,
           eW0, eb0, eWh, ebh, eWs, ebs, eWo, ebo):
    eidx, gval = _gate(x, latent, gW1, gb1, gW2, gb2, gln_g, gln_b, gW3, gb3)
    xl = jnp.concatenate([x, latent], axis=1)
    xls, gs, slots, bexp = _route(eidx, gval, xl)
    fs = _grouped_mlp(bexp, xls, gs.reshape(P, 1),
                      eW0, eb0.reshape(E, 1, HID), eWh, ebh, eWs,
                      ebs.reshape(E, 1, HID), eWo, ebo.reshape(E, 1, OUT))
    out = _combine(fs, slots)
    return out[:, :OUT]
